# baseline (speedup 1.0000x reference)
import jax
import jax.numpy as jnp
from jax.experimental import pallas as pl
from jax.experimental.pallas import tpu as pltpu

_B, _T = 16, 8
_BT = _B * _T
_DA, _DT = 768, 512
_V = 49408
_VT = 256
_NT = _V // _VT
_TAU = 0.1
_STD_SCALE = 1.0


def _stats_body(emb_ref, out_ref):
    i = pl.program_id(0)
    e = emb_ref[...]
    s = jnp.sum(e, axis=0, keepdims=True)
    s2 = jnp.sum(e * e, axis=0, keepdims=True)
    tile = jnp.concatenate([s, s2], axis=0)

    @pl.when(i == 0)
    def _init():
        out_ref[...] = tile

    @pl.when(i > 0)
    def _acc():
        out_ref[...] += tile


def _flash_body(audio_ref, w_ref, b_ref, stats_ref, emb_ref,
                cos_ref, kw_ref, fn_ref, acc_ref, l_ref):
    i = pl.program_id(0)

    @pl.when(i == 0)
    def _prologue():
        stats = stats_ref[...]
        emb_mean = stats[0:1, :] / _V
        emb_ex2 = stats[1:2, :] / _V
        emb_std = jnp.sqrt(jnp.maximum(emb_ex2 - emb_mean * emb_mean, 0.0))
        feats = jnp.dot(audio_ref[...], w_ref[...],
                        preferred_element_type=jnp.float32) + b_ref[...]
        mu = jnp.mean(feats, axis=0, keepdims=True)
        var = jnp.mean((feats - mu) * (feats - mu), axis=0, keepdims=True)
        normed = (feats - mu) / jnp.sqrt(var + 1e-5)
        f = normed * (emb_std * _STD_SCALE) + emb_mean
        norm = jnp.sqrt(jnp.sum(f * f, axis=1, keepdims=True)) + 1e-8
        fn_ref[...] = f / norm
        acc_ref[...] = jnp.zeros((_BT, _DT), jnp.float32)
        l_ref[...] = jnp.zeros((_BT, 1), jnp.float32)

    e = emb_ref[...]
    e_norm = jnp.sqrt(jnp.sum(e * e, axis=1, keepdims=True)) + 1e-8
    fn = fn_ref[...]
    cos = jax.lax.dot_general(fn, e, (((1,), (1,)), ((), ())),
                              preferred_element_type=jnp.float32)
    cos = cos / e_norm.T
    cos_ref[...] = cos
    p = jnp.exp(cos * (1.0 / _TAU))
    l_ref[...] += jnp.sum(p, axis=1, keepdims=True)
    acc_ref[...] += jnp.dot(p, e, preferred_element_type=jnp.float32)

    @pl.when(i == _NT - 1)
    def _epilogue():
        kw_ref[...] = acc_ref[...] / l_ref[...]


def kernel(audio_feat, W_proj, b_proj, token_embedding):
    audio2d = audio_feat.reshape(_BT, _DA)
    b2d = b_proj.reshape(1, _DT)

    stats = pl.pallas_call(
        _stats_body,
        grid=(_NT,),
        in_specs=[pl.BlockSpec((_VT, _DT), lambda i: (i, 0))],
        out_specs=pl.BlockSpec((2, _DT), lambda i: (0, 0)),
        out_shape=jax.ShapeDtypeStruct((2, _DT), jnp.float32),
    )(token_embedding)

    cos_score, keywords = pl.pallas_call(
        _flash_body,
        grid=(_NT,),
        in_specs=[
            pl.BlockSpec((_BT, _DA), lambda i: (0, 0)),
            pl.BlockSpec((_DA, _DT), lambda i: (0, 0)),
            pl.BlockSpec((1, _DT), lambda i: (0, 0)),
            pl.BlockSpec((2, _DT), lambda i: (0, 0)),
            pl.BlockSpec((_VT, _DT), lambda i: (i, 0)),
        ],
        out_specs=[
            pl.BlockSpec((_BT, _VT), lambda i: (0, i)),
            pl.BlockSpec((_BT, _DT), lambda i: (0, 0)),
        ],
        out_shape=[
            jax.ShapeDtypeStruct((_BT, _V), jnp.float32),
            jax.ShapeDtypeStruct((_BT, _DT), jnp.float32),
        ],
        scratch_shapes=[
            pltpu.VMEM((_BT, _DT), jnp.float32),
            pltpu.VMEM((_BT, _DT), jnp.float32),
            pltpu.VMEM((_BT, 1), jnp.float32),
        ],
    )(audio2d, W_proj, b2d, stats, token_embedding)

    return (keywords.reshape(_B, _T, _DT), cos_score.reshape(_B, _T, _V))
